# speedup vs baseline: 230.0223x; 230.0223x over previous
import functools

import jax
import jax.numpy as jnp
from jax import lax
from jax.experimental import pallas as pl


def _fused_body(topk, x_ref, w_ref, b_ref, emb_ref, y_ref):
    n = emb_ref.shape[0]
    batch = x_ref.shape[0]
    f32 = jnp.float32

    emb = emb_ref[...]
    gram = lax.dot_general(emb, emb, (((1,), (1,)), ((), ())),
                           preferred_element_type=f32)
    nrm = jnp.sqrt(jnp.sum(emb * emb, axis=1))
    cos = gram / (nrm[:, None] * nrm[None, :] + 1e-8)

    chunk = 8
    ranks = []
    for c in range(n // chunk):
        ch = cos[c * chunk:(c + 1) * chunk, :]
        at_l = ch[:, None, :]
        at_j = ch[:, :, None]
        li = lax.broadcasted_iota(jnp.int32, (chunk, n, n), 2)
        ji = lax.broadcasted_iota(jnp.int32, (chunk, n, n), 1)
        sel = (at_l > at_j) | ((at_l == at_j) & (li < ji))
        ranks.append(jnp.sum(sel.astype(f32), axis=2))
    rank = jnp.concatenate(ranks, axis=0)
    t_mask = (rank < topk).astype(f32)

    sym = jnp.minimum(t_mask + t_mask.T, 1.0)
    ii = lax.broadcasted_iota(jnp.int32, (n, n), 0)
    jj = lax.broadcasted_iota(jnp.int32, (n, n), 1)
    eye = (ii == jj).astype(f32)
    nbr = jnp.minimum(sym + eye, 1.0)
    common = lax.dot_general(nbr, nbr, (((1,), (1,)), ((), ())),
                             preferred_element_type=f32)
    maxc = jnp.max(common)
    edge_mask = sym * (common > 1.0).astype(f32)
    coeff = jnp.where(edge_mask > 0, common * common / maxc, 0.0)
    tc = t_mask * coeff
    deg = jnp.sum(tc, axis=1)
    dinv = jnp.where(deg > 0, 1.0 / jnp.sqrt(deg), 0.0)
    a_mat = tc * (dinv[:, None] * dinv[None, :])

    w = w_ref[...]
    bias_col = b_ref[...]
    for b in range(batch):
        xb = x_ref[b]
        wx = lax.dot_general(w, xb, (((0,), (0,)), ((), ())),
                             preferred_element_type=f32)
        yb = lax.dot_general(wx, a_mat, (((1,), (1,)), ((), ())),
                             preferred_element_type=f32)
        y_ref[b] = yb + bias_col


def kernel(x, weight, bias, embedding):
    batch, seq, n = x.shape
    topk = int(0.3 * n)
    body = functools.partial(_fused_body, topk)
    out = pl.pallas_call(
        body,
        out_shape=jax.ShapeDtypeStruct((batch, seq, n), jnp.float32),
    )(x, weight, bias.reshape(seq, 1), embedding)
    return out
